# baseline (speedup 1.0000x reference)
import jax
import jax.numpy as jnp
from jax.experimental import pallas as pl
from jax.experimental.pallas import tpu as pltpu

B, C, H, W = 32, 768, 32, 32
NROWS = (B * C * H * W) // 128
CHUNK = 2048
NCHUNK = NROWS // CHUNK
NBUF = 8
BIAS_ROWS = C * 8
BIAS_PERIOD = BIAS_ROWS // CHUNK


def _stream_kernel(x_hbm, ht_v, vt_v, out_hbm,
                   inb, outb, bias_ref, in_sems, out_sems):
    htT = ht_v[...].T
    vtT = vt_v[...].T
    for s in range(8):
        pieces = [htT[:, 4 * s + k:4 * s + k + 1] + vtT for k in range(4)]
        bias_ref[:, s, :] = jnp.concatenate(pieces, axis=1)
    bias2 = bias_ref.reshape(BIAS_ROWS, 128)

    def start_in(i, k):
        pltpu.make_async_copy(
            x_hbm.at[pl.ds(i * CHUNK, CHUNK)], inb.at[k], in_sems.at[k]
        ).start()

    for k in range(NBUF):
        start_in(k, k)

    def loop_body(i, carry):
        k = jax.lax.rem(i, NBUF)
        pltpu.make_async_copy(
            x_hbm.at[pl.ds(i * CHUNK, CHUNK)], inb.at[k], in_sems.at[k]
        ).wait()

        @pl.when(i >= NBUF)
        def _():
            pltpu.make_async_copy(
                outb.at[k], out_hbm.at[pl.ds(0, CHUNK)], out_sems.at[k]
            ).wait()

        off = jax.lax.rem(i, BIAS_PERIOD) * CHUNK
        outb[k] = inb[k] + bias2[pl.ds(off, CHUNK)]
        pltpu.make_async_copy(
            outb.at[k], out_hbm.at[pl.ds(i * CHUNK, CHUNK)], out_sems.at[k]
        ).start()

        @pl.when(i + NBUF < NCHUNK)
        def _():
            start_in(i + NBUF, k)

        return carry

    jax.lax.fori_loop(0, NCHUNK, loop_body, 0)

    for i in range(NCHUNK - NBUF, NCHUNK):
        k = i % NBUF
        pltpu.make_async_copy(
            outb.at[k], out_hbm.at[pl.ds(0, CHUNK)], out_sems.at[k]
        ).wait()


def kernel(x, register_table, vertical_table, horizontal_table):
    Bb, Cc, Hh, Ww = x.shape
    x2 = x.reshape((Bb * Cc * Hh * Ww) // 128, 128)

    out2 = pl.pallas_call(
        _stream_kernel,
        grid=(1,),
        in_specs=[
            pl.BlockSpec(memory_space=pl.ANY),
            pl.BlockSpec(horizontal_table.shape, lambda i: (0, 0)),
            pl.BlockSpec(vertical_table.shape, lambda i: (0, 0)),
        ],
        out_specs=pl.BlockSpec(memory_space=pl.ANY),
        out_shape=jax.ShapeDtypeStruct(x2.shape, x.dtype),
        scratch_shapes=[
            pltpu.VMEM((NBUF, CHUNK, 128), jnp.float32),
            pltpu.VMEM((NBUF, CHUNK, 128), jnp.float32),
            pltpu.VMEM((C, 8, 128), jnp.float32),
            pltpu.SemaphoreType.DMA((NBUF,)),
            pltpu.SemaphoreType.DMA((NBUF,)),
        ],
    )(x2, horizontal_table, vertical_table)

    return (out2.reshape(x.shape), register_table)


# speedup vs baseline: 1.0042x; 1.0042x over previous
import jax
import jax.numpy as jnp
from jax.experimental import pallas as pl
from jax.experimental.pallas import tpu as pltpu

B, C, H, W = 32, 768, 32, 32
CHUNK = C * 8
NCHUNK = B
NBUF = 6


def _stream_kernel(x_raw, ht_v, vt_v, out_raw,
                   inb, outb, bias_ref, in_sems, out_sems):
    htT = ht_v[...].T
    vtT = vt_v[...].T
    for s in range(8):
        pieces = [htT[:, 4 * s + k:4 * s + k + 1] + vtT for k in range(4)]
        bias_ref[:, s, :] = jnp.concatenate(pieces, axis=1)
    bias2 = bias_ref.reshape(CHUNK, 128)

    def start_in(i, k):
        pltpu.make_async_copy(
            x_raw.at[pl.ds(i * CHUNK, CHUNK)], inb.at[k], in_sems.at[k]
        ).start()

    for k in range(NBUF):
        start_in(k, k)

    for i in range(NCHUNK):
        k = i % NBUF
        pltpu.make_async_copy(
            x_raw.at[pl.ds(i * CHUNK, CHUNK)], inb.at[k], in_sems.at[k]
        ).wait()
        if i >= NBUF:
            pltpu.make_async_copy(
                outb.at[k], out_raw.at[pl.ds(0, CHUNK)], out_sems.at[k]
            ).wait()
        outb[k] = inb[k] + bias2[...]
        pltpu.make_async_copy(
            outb.at[k], out_raw.at[pl.ds(i * CHUNK, CHUNK)], out_sems.at[k]
        ).start()
        if i + NBUF < NCHUNK:
            start_in(i + NBUF, k)

    for i in range(NCHUNK - NBUF, NCHUNK):
        k = i % NBUF
        pltpu.make_async_copy(
            outb.at[k], out_raw.at[pl.ds(0, CHUNK)], out_sems.at[k]
        ).wait()


def kernel(x, register_table, vertical_table, horizontal_table):
    Bb, Cc, Hh, Ww = x.shape
    x2 = x.reshape((Bb * Cc * Hh * Ww) // 128, 128)
    out2 = pl.pallas_call(
        _stream_kernel,
        grid=(1,),
        in_specs=[
            pl.BlockSpec(memory_space=pl.ANY),
            pl.BlockSpec(horizontal_table.shape, lambda i: (0, 0)),
            pl.BlockSpec(vertical_table.shape, lambda i: (0, 0)),
        ],
        out_specs=pl.BlockSpec(memory_space=pl.ANY),
        out_shape=jax.ShapeDtypeStruct(x2.shape, x.dtype),
        scratch_shapes=[
            pltpu.VMEM((NBUF, CHUNK, 128), jnp.float32),
            pltpu.VMEM((NBUF, CHUNK, 128), jnp.float32),
            pltpu.VMEM((C, 8, 128), jnp.float32),
            pltpu.SemaphoreType.DMA((NBUF,)),
            pltpu.SemaphoreType.DMA((NBUF,)),
        ],
    )(x2, horizontal_table, vertical_table)

    return (out2.reshape(x.shape), register_table)


# speedup vs baseline: 1.4454x; 1.4393x over previous
import jax
import jax.numpy as jnp
from jax.experimental import pallas as pl
from jax.experimental.pallas import tpu as pltpu

B, C, H, W = 32, 768, 32, 32
NSPLIT = 4
QROWS = C // NSPLIT


def _bias_add_kernel(inA, inB, inC, inD, ht_ref, vt_ref, out_ref, bias_ref):
    @pl.when(pl.program_id(0) == 0)
    def _():
        htT = ht_ref[...].T
        vtT = vt_ref[...].T
        for s in range(8):
            pieces = [htT[:, 4 * s + k:4 * s + k + 1] + vtT for k in range(4)]
            bias_ref[:, s, :] = jnp.concatenate(pieces, axis=1)

    for q, ref in enumerate((inA, inB, inC, inD)):
        lo = q * QROWS
        out_ref[lo:lo + QROWS] = ref[...] + bias_ref[lo:lo + QROWS]


def kernel(x, register_table, vertical_table, horizontal_table):
    Bb, Cc, Hh, Ww = x.shape
    x3 = x.reshape(Bb * Cc, (Hh * Ww) // 128, 128)

    def make_qspec(q):
        return pl.BlockSpec((QROWS, 8, 128), lambda i: (i * NSPLIT + q, 0, 0))

    out3 = pl.pallas_call(
        _bias_add_kernel,
        grid=(Bb,),
        in_specs=[make_qspec(q) for q in range(NSPLIT)] + [
            pl.BlockSpec(horizontal_table.shape, lambda i: (0, 0)),
            pl.BlockSpec(vertical_table.shape, lambda i: (0, 0)),
        ],
        out_specs=pl.BlockSpec((Cc, 8, 128), lambda i: (i, 0, 0)),
        out_shape=jax.ShapeDtypeStruct(x3.shape, x.dtype),
        scratch_shapes=[pltpu.VMEM((Cc, 8, 128), jnp.float32)],
    )(x3, x3, x3, x3, horizontal_table, vertical_table)

    return (out3.reshape(x.shape), register_table)


# speedup vs baseline: 3.3893x; 2.3449x over previous
import jax
import jax.numpy as jnp
from jax.experimental import pallas as pl
from jax.experimental.pallas import tpu as pltpu

B, C, H, W = 32, 768, 32, 32
BBLK = 4


def _bias_add_kernel(x_ref, ht_ref, vt_ref, out_ref, bias_ref):
    @pl.when(pl.program_id(0) == 0)
    def _():
        htT = ht_ref[...].T
        vtT = vt_ref[...].T
        for s in range(8):
            pieces = [htT[:, 4 * s + k:4 * s + k + 1] + vtT for k in range(4)]
            bias_ref[:, s, :] = jnp.concatenate(pieces, axis=1)

    for b in range(BBLK):
        out_ref[b] = x_ref[b] + bias_ref[...]


def kernel(x, register_table, vertical_table, horizontal_table):
    Bb, Cc, Hh, Ww = x.shape
    x3 = x.reshape(Bb, Cc, (Hh * Ww) // 128, 128)

    out3 = pl.pallas_call(
        _bias_add_kernel,
        grid=(Bb // BBLK,),
        in_specs=[
            pl.BlockSpec((BBLK, Cc, 8, 128), lambda i: (i, 0, 0, 0)),
            pl.BlockSpec(horizontal_table.shape, lambda i: (0, 0)),
            pl.BlockSpec(vertical_table.shape, lambda i: (0, 0)),
        ],
        out_specs=pl.BlockSpec((BBLK, Cc, 8, 128), lambda i: (i, 0, 0, 0)),
        out_shape=jax.ShapeDtypeStruct(x3.shape, x.dtype),
        scratch_shapes=[pltpu.VMEM((Cc, 8, 128), jnp.float32)],
    )(x3, horizontal_table, vertical_table)

    return (out3.reshape(x.shape), register_table)
